# baseline (speedup 1.0000x reference)
import functools

import jax
import jax.numpy as jnp
from jax import lax
from jax.experimental import pallas as pl
from jax.experimental.pallas import tpu as pltpu
from jax.experimental.pallas import tpu_sc as plsc

NB = 4096
NT = 200
DIM = 32
NC = 2
NS = 16
NW = NC * NS
G = 128
K = 10
N_OUTER = NT // K

_mesh = plsc.VectorSubcoreMesh(core_axis_name="c", subcore_axis_name="s")


@functools.partial(
    pl.kernel,
    out_type=jax.ShapeDtypeStruct((NT, NB, DIM), jnp.float32),
    mesh=_mesh,
    compiler_params=pltpu.CompilerParams(use_tc_tiling_on_sc=False),
    scratch_types=[
        pltpu.VMEM((NT, G), jnp.int32),
        pltpu.VMEM((K, G, DIM), jnp.float32),
        pltpu.VMEM((K, G, DIM), jnp.float32),
        pltpu.SemaphoreType.DMA,
        pltpu.SemaphoreType.DMA,
        pltpu.SemaphoreType.DMA,
        pltpu.SemaphoreType.DMA,
    ],
)
def _gather_kernel(idx_hbm, table_hbm, out_hbm, idx_all, rows0, rows1,
                   sg0, sg1, ss0, ss1):
    wid = lax.axis_index("s") * NC + lax.axis_index("c")
    b0 = wid * G

    pltpu.sync_copy(idx_hbm.at[:, pl.ds(b0, G)], idx_all)

    bufs = ((rows0, sg0, ss0), (rows1, sg1, ss1))

    def fire_gathers(c, buf, sem):
        for j in range(K):
            pltpu.async_copy(
                table_hbm.at[idx_all.at[c * K + j]],
                buf.at[j],
                sem,
            )

    def wait_gathers(buf, sem):
        pltpu.make_async_copy(out_hbm.at[pl.ds(0, K), pl.ds(0, G)], buf,
                              sem).wait()

    def store(c, buf, sem):
        return pltpu.async_copy(
            buf, out_hbm.at[pl.ds(c * K, K), pl.ds(b0, G)], sem)

    fire_gathers(0, rows0, sg0)
    fire_gathers(1, rows1, sg1)

    @pl.loop(0, N_OUTER - 2, step=2)
    def _pipe(i):
        for b in range(2):
            c = i + b
            buf, sg, ss = bufs[b]
            wait_gathers(buf, sg)
            store(c, buf, ss).wait()
            fire_gathers(c + 2, buf, sg)

    for b in range(2):
        c = N_OUTER - 2 + b
        buf, sg, ss = bufs[b]
        wait_gathers(buf, sg)
        store(c, buf, ss).wait()


def kernel(token_ids, embedding_mat):
    out_t = _gather_kernel(token_ids.T, embedding_mat)
    return out_t.transpose(1, 0, 2)
